# Initial kernel scaffold; baseline (speedup 1.0000x reference)
import jax
import jax.numpy as jnp
from jax.experimental import pallas as pl


def kernel(x):
    raise NotImplementedError("write your pallas kernel here")


# speedup vs baseline: 474.5108x; 474.5108x over previous
import math

import jax
import jax.numpy as jnp
import numpy as np
from jax.experimental import pallas as pl

_N_FFT = 1024
_HOP = 256
_T = 131072
_K = _N_FFT // 2 + 1
_F = _T // _HOP + 1
_FP = 520
_KP = 640
_ROWS = _FP + 3


def _twiddles():
    n = np.arange(_N_FFT, dtype=np.float64)
    k = np.arange(_KP, dtype=np.float64)
    ang = 2.0 * np.pi * np.outer(n, k % _N_FFT) / _N_FFT
    win = 0.5 * (1.0 - np.cos(2.0 * np.pi * n / _N_FFT))
    ct = np.cos(ang) * win[:, None]
    st = np.sin(ang) * win[:, None]
    ct[:, _K:] = 0.0
    st[:, _K:] = 0.0
    return jnp.asarray(ct, jnp.float32), jnp.asarray(st, jnp.float32)


def _stft_kernel(xr_ref, ct_ref, st_ref, outr_ref, outi_ref):
    ct = ct_ref[...]
    st = st_ref[...]
    rr = xr_ref[0]
    ri = xr_ref[1]
    fr = jnp.concatenate([rr[a:a + _FP, :] for a in range(4)], axis=1)
    fi = jnp.concatenate([ri[a:a + _FP, :] for a in range(4)], axis=1)
    out_r = (jnp.dot(fr, ct, preferred_element_type=jnp.float32)
             + jnp.dot(fi, st, preferred_element_type=jnp.float32))
    out_i = (jnp.dot(fi, ct, preferred_element_type=jnp.float32)
             - jnp.dot(fr, st, preferred_element_type=jnp.float32))
    outr_ref[...] = out_r.T
    outi_ref[...] = out_i.T


@jax.jit
def kernel(x):
    ct, st = _twiddles()
    xpad = jnp.pad(x, ((0, 0), (_N_FFT // 2, _N_FFT // 2)), mode="reflect")
    xpad = jnp.pad(xpad, ((0, 0), (0, _ROWS * _HOP - xpad.shape[1])))
    xr = xpad.reshape(2, _ROWS, _HOP)
    out_r, out_i = pl.pallas_call(
        _stft_kernel,
        out_shape=(
            jax.ShapeDtypeStruct((_KP, _FP), jnp.float32),
            jax.ShapeDtypeStruct((_KP, _FP), jnp.float32),
        ),
    )(xr, ct, st)
    return jnp.stack([out_r[:_K, :_F], out_i[:_K, :_F]], axis=0)
